# baseline (speedup 1.0000x reference)
import math

import jax
import jax.numpy as jnp
from jax import lax
from jax.experimental import pallas as pl
from jax.experimental.pallas import tpu as pltpu
from jax.experimental.pallas import tpu_sc as plsc

N = 10000
E = 160000
D = 256
H = 128
B = 100
WIN = 10

NP = 10240
NC, NS = 2, 16
NW = NC * NS
CH = 128
EP = 163840
EPT = EP // NW
NCH = EPT // CH
ACC_ROWS = NP + 256
ZR = ACC_ROWS // NS
OUTR = NP // NS
TCH = EP // CH // NS
BM = 1024
BG = 512
RSQ = 1.0 / math.sqrt(1.0 + 1e-5)

_MESH = dict(core_axis_name="c", subcore_axis_name="s", num_cores=NC,
             num_subcores=NS)


def _zero_acc(acc, zb, zhbm, semz, s):
    pltpu.sync_copy(zhbm, zb)
    pend = []
    for zi in range(ZR // 16):
        pend.append(pltpu.async_copy(
            zb, acc.at[pl.ds(s * ZR + zi * 16, 16)], semz))
    for d in pend:
        d.wait()


def _edge_loop(xn, src3, dst3, acc, sl0, sl1, dl0, dl1, r0, r1,
               sem0, sem1, semi0, semi1, s):
    g0 = s * TCH

    def idx_load(j, sl, dl, semi):
        pltpu.async_copy(src3.at[g0 + j], sl, semi)
        pltpu.async_copy(dst3.at[g0 + j], dl, semi)

    def idx_wait(j, sl, dl, semi):
        pltpu.make_async_copy(src3.at[g0 + j], sl, semi).wait()
        pltpu.make_async_copy(dst3.at[g0 + j], dl, semi).wait()

    idx_load(0, sl0, dl0, semi0)
    idx_load(1, sl1, dl1, semi1)
    idx_wait(0, sl0, dl0, semi0)
    pltpu.async_copy(xn.at[sl0], r0, sem0)

    def step(it, carry):
        i0 = it * 2
        idx_wait(i0 + 1, sl1, dl1, semi1)
        pltpu.async_copy(xn.at[sl1], r1, sem1)
        pltpu.make_async_copy(xn.at[sl0], r0, sem0).wait()
        pltpu.sync_copy(r0, acc.at[dl0], add=True)

        @pl.when(i0 + 2 < TCH)
        def _():
            idx_load(i0 + 2, sl0, dl0, semi0)

        pltpu.make_async_copy(xn.at[sl1], r1, sem1).wait()
        pltpu.sync_copy(r1, acc.at[dl1], add=True)

        @pl.when(i0 + 3 < TCH)
        def _():
            idx_load(i0 + 3, sl1, dl1, semi1)

        @pl.when(i0 + 2 < TCH)
        def _():
            idx_wait(i0 + 2, sl0, dl0, semi0)
            pltpu.async_copy(xn.at[sl0], r0, sem0)

        return carry

    lax.fori_loop(0, TCH // 2, step, 0)


def _deg_loop(dst3, ones_hbm, acc, di, r0, semd, s):
    pltpu.sync_copy(dst3.at[pl.ds(s * TCH, TCH)], di)
    pltpu.sync_copy(ones_hbm, r0)

    def step(it, carry):
        i0 = it * 2
        pltpu.async_copy(r0, acc.at[di.at[i0]], semd, add=True)
        pltpu.async_copy(r0, acc.at[di.at[i0 + 1]], semd, add=True)
        pltpu.make_async_copy(r0, acc.at[di.at[i0]], semd).wait()
        pltpu.make_async_copy(r0, acc.at[di.at[i0 + 1]], semd).wait()
        return carry

    lax.fori_loop(0, TCH // 2, step, 0)


def _make_sc_body(with_deg):
    def body(xn, src3, dst3, ones_hbm, zhbm, out, acc, zb, sl0, sl1, dl0,
             dl1, di, r0, r1, sem0, sem1, semi0, semi1, semz):
        c = lax.axis_index("c")
        s = lax.axis_index("s")
        zero_cores = NC if with_deg else 1

        @pl.when(c < zero_cores)
        def _():
            _zero_acc(acc, zb, zhbm, semz, s)
        plsc.subcore_barrier()

        @pl.when(c == 0)
        def _():
            _edge_loop(xn, src3, dst3, acc, sl0, sl1, dl0, dl1, r0, r1,
                       sem0, sem1, semi0, semi1, s)
        if with_deg:
            @pl.when(c == 1)
            def _():
                _deg_loop(dst3, ones_hbm, acc, di, r0, semz, s)
        plsc.subcore_barrier()

        @pl.when(c < zero_cores)
        def _():
            pltpu.sync_copy(acc.at[pl.ds(s * OUTR, OUTR)],
                            out.at[c, pl.ds(s * OUTR, OUTR)])

    return body


def _sc_scatter(xn, src3, dst3, ones_hbm, zhbm, with_deg):
    return pl.kernel(
        _make_sc_body(with_deg),
        out_type=jax.ShapeDtypeStruct((NC if with_deg else 1, NP, H),
                                      jnp.float32),
        mesh=plsc.VectorSubcoreMesh(**_MESH),
        scratch_types=[
            pltpu.VMEM_SHARED((ACC_ROWS, H), jnp.float32),
            pltpu.VMEM((16, H), jnp.float32),
            pltpu.VMEM((CH,), jnp.int32),
            pltpu.VMEM((CH,), jnp.int32),
            pltpu.VMEM((CH,), jnp.int32),
            pltpu.VMEM((CH,), jnp.int32),
            pltpu.VMEM((TCH, CH), jnp.int32),
            pltpu.VMEM((CH, H), jnp.float32),
            pltpu.VMEM((CH, H), jnp.float32),
            pltpu.SemaphoreType.DMA,
            pltpu.SemaphoreType.DMA,
            pltpu.SemaphoreType.DMA,
            pltpu.SemaphoreType.DMA,
            pltpu.SemaphoreType.DMA,
        ],
    )(xn, src3, dst3, ones_hbm, zhbm)


def _mm2_body(x_ref, ws_ref, wn_ref, xs_ref, xn_ref):
    x = x_ref[...]
    xs_ref[...] = jnp.dot(x, ws_ref[...], preferred_element_type=jnp.float32)
    xn_ref[...] = jnp.dot(x, wn_ref[...], preferred_element_type=jnp.float32)


def _mm2(x, ws, wn):
    n, k = x.shape
    return pl.pallas_call(
        _mm2_body,
        grid=(n // BM,),
        in_specs=[
            pl.BlockSpec((BM, k), lambda i: (i, 0)),
            pl.BlockSpec((k, H), lambda i: (0, 0)),
            pl.BlockSpec((k, H), lambda i: (0, 0)),
        ],
        out_specs=[
            pl.BlockSpec((BM, H), lambda i: (i, 0)),
            pl.BlockSpec((BM, H), lambda i: (i, 0)),
        ],
        out_shape=[jax.ShapeDtypeStruct((n, H), jnp.float32)] * 2,
    )(x, ws, wn)


def _ew_math(xs_ref, p_ref, degp_ref, b_ref, a_ref, g_ref, be_ref):
    deg = degp_ref[0, :, 0:1]
    inv = 1.0 / jnp.maximum(deg, 1.0)
    t = xs_ref[...] + p_ref[0, :, :] * inv + b_ref[...]
    t = jnp.where(t >= 0.0, t, a_ref[...] * t)
    return g_ref[...] * t * RSQ + be_ref[...]


def _ew_body(xs_ref, p_ref, degp_ref, b_ref, a_ref, g_ref, be_ref, h_ref):
    h_ref[...] = _ew_math(xs_ref, p_ref, degp_ref, b_ref, a_ref, g_ref,
                          be_ref)


def _ewmm_body(xs_ref, p_ref, degp_ref, b_ref, a_ref, g_ref, be_ref, ws_ref,
               wn_ref, h_ref, xs2_ref, xn2_ref):
    h = _ew_math(xs_ref, p_ref, degp_ref, b_ref, a_ref, g_ref, be_ref)
    h_ref[...] = h
    xs2_ref[...] = jnp.dot(h, ws_ref[...], preferred_element_type=jnp.float32)
    xn2_ref[...] = jnp.dot(h, wn_ref[...], preferred_element_type=jnp.float32)


def _ewmm(xs, p, degp, b, a, g, be, ws, wn):
    return pl.pallas_call(
        _ewmm_body,
        grid=(NP // BM,),
        in_specs=[
            pl.BlockSpec((BM, H), lambda i: (i, 0)),
            pl.BlockSpec((1, BM, H), lambda i: (0, i, 0)),
            pl.BlockSpec((1, BM, H), lambda i: (1, i, 0)),
            pl.BlockSpec((1, H), lambda i: (0, 0)),
            pl.BlockSpec((1, H), lambda i: (0, 0)),
            pl.BlockSpec((1, H), lambda i: (0, 0)),
            pl.BlockSpec((1, H), lambda i: (0, 0)),
            pl.BlockSpec((H, H), lambda i: (0, 0)),
            pl.BlockSpec((H, H), lambda i: (0, 0)),
        ],
        out_specs=[
            pl.BlockSpec((BM, H), lambda i: (i, 0)),
            pl.BlockSpec((BM, H), lambda i: (i, 0)),
            pl.BlockSpec((BM, H), lambda i: (i, 0)),
        ],
        out_shape=[jax.ShapeDtypeStruct((NP, H), jnp.float32)] * 3,
    )(xs, p, degp, b, a, g, be, ws, wn)


def _ew(xs, p, degp, b, a, g, be):
    return pl.pallas_call(
        _ew_body,
        grid=(NP // BM,),
        in_specs=[
            pl.BlockSpec((BM, H), lambda i: (i, 0)),
            pl.BlockSpec((1, BM, H), lambda i: (0, i, 0)),
            pl.BlockSpec((1, BM, H), lambda i: (1, i, 0)),
            pl.BlockSpec((1, H), lambda i: (0, 0)),
            pl.BlockSpec((1, H), lambda i: (0, 0)),
            pl.BlockSpec((1, H), lambda i: (0, 0)),
            pl.BlockSpec((1, H), lambda i: (0, 0)),
        ],
        out_specs=pl.BlockSpec((BM, H), lambda i: (i, 0)),
        out_shape=jax.ShapeDtypeStruct((NP, H), jnp.float32),
    )(xs, p, degp, b, a, g, be)


def _graph_body(gid_ref, h1_ref, h2_ref, h3_ref, h4_ref, wl_ref, bl_ref,
                out_ref, gs_scr):
    j = pl.program_id(0)

    @pl.when(j == 0)
    def _():
        gs_scr[...] = jnp.zeros_like(gs_scr)

    gid = gid_ref[0, 0, :]
    onehot = (lax.broadcasted_iota(jnp.int32, (128, BG), 0)
              == gid[None, :]).astype(jnp.float32)
    hcat = jnp.concatenate(
        [h1_ref[...], h2_ref[...], h3_ref[...], h4_ref[...],
         jnp.ones((BG, 128), jnp.float32)], axis=1)
    gs_scr[...] += jnp.dot(onehot, hcat, preferred_element_type=jnp.float32)

    @pl.when(j == pl.num_programs(0) - 1)
    def _():
        cnt = gs_scr[:, 512:513]
        gvec = gs_scr[:, :512] / jnp.maximum(cnt, 1.0)
        out_ref[...] = (jnp.dot(gvec, wl_ref[...],
                                preferred_element_type=jnp.float32)
                        + bl_ref[...])


def _graph(gid, h1, h2, h3, h4, wl, bl):
    return pl.pallas_call(
        _graph_body,
        grid=(NP // BG,),
        in_specs=[
            pl.BlockSpec((1, 1, BG), lambda i: (i, 0, 0)),
            pl.BlockSpec((BG, H), lambda i: (i, 0)),
            pl.BlockSpec((BG, H), lambda i: (i, 0)),
            pl.BlockSpec((BG, H), lambda i: (i, 0)),
            pl.BlockSpec((BG, H), lambda i: (i, 0)),
            pl.BlockSpec((4 * H, WIN * H), lambda i: (0, 0)),
            pl.BlockSpec((1, WIN * H), lambda i: (0, 0)),
        ],
        out_specs=pl.BlockSpec((128, WIN * H), lambda i: (0, 0)),
        out_shape=jax.ShapeDtypeStruct((128, WIN * H), jnp.float32),
        scratch_shapes=[pltpu.VMEM((128, 5 * H), jnp.float32)],
    )(gid, h1, h2, h3, h4, wl, bl)


def kernel(x, edge_index, graph_ids, Ws0, Wn0, b0, a0, g0, be0, Ws1, Wn1, b1,
           a1, g1, be1, Ws2, Wn2, b2, a2, g2, be2, Ws3, Wn3, b3, a3, g3, be3,
           Wl, bl):
    f32 = jnp.float32
    src = edge_index[0]
    dst = edge_index[1]
    src3 = jnp.concatenate(
        [src, jnp.full((EP - E,), N - 1, jnp.int32)]).reshape(EP // CH, CH)
    dst3 = jnp.concatenate(
        [dst, jnp.full((EP - E,), NP, jnp.int32)]).reshape(EP // CH, CH)
    xp = jnp.pad(x, ((0, NP - N), (0, 0)))
    gid = jnp.concatenate(
        [graph_ids, jnp.full((NP - N,), 127, jnp.int32)]).reshape(
            NP // BG, 1, BG)
    zhbm = jnp.zeros((16, H), f32)
    ones_hbm = jnp.ones((CH, H), f32)

    params = [(Ws0, Wn0, b0, a0, g0, be0), (Ws1, Wn1, b1, a1, g1, be1),
              (Ws2, Wn2, b2, a2, g2, be2), (Ws3, Wn3, b3, a3, g3, be3)]
    hs = []
    xs, xn = _mm2(xp, params[0][0], params[0][1])
    degp = None
    for li, (ws, wn, b, a, g, be) in enumerate(params):
        p = _sc_scatter(xn, src3, dst3, ones_hbm, zhbm, li == 0)
        if li == 0:
            degp = p
        args = (xs, p, degp, b.reshape(1, H), a.reshape(1, H),
                g.reshape(1, H), be.reshape(1, H))
        if li < 3:
            h, xs, xn = _ewmm(*args, params[li + 1][0], params[li + 1][1])
        else:
            h = _ew(*args)
        hs.append(h)

    outp = _graph(gid, hs[0], hs[1], hs[2], hs[3], Wl,
                  bl.reshape(1, WIN * H))
    return outp[:B].reshape(B, WIN, H)


# speedup vs baseline: 1.1906x; 1.1906x over previous
import math

import jax
import jax.numpy as jnp
from jax import lax
from jax.experimental import pallas as pl
from jax.experimental.pallas import tpu as pltpu
from jax.experimental.pallas import tpu_sc as plsc

N = 10000
E = 160000
D = 256
H = 128
B = 100
WIN = 10

NP = 10240
NC, NS = 2, 16
NW = NC * NS
CH = 128
EP = 163840
EPT = EP // NW
NCH = EPT // CH
ACC_ROWS = NP + 256
ZR = ACC_ROWS // NS
OUTR = NP // NS
F0 = 1248
NCH0 = F0 // NS
NCH1 = (EP // CH - F0) // NS
BM = 1024
BG = 512
RSQ = 1.0 / math.sqrt(1.0 + 1e-5)

_MESH = dict(core_axis_name="c", subcore_axis_name="s", num_cores=NC,
             num_subcores=NS)


def _zero_acc(acc, zb, zhbm, semz, s):
    pltpu.sync_copy(zhbm, zb)
    pend = []
    for zi in range(ZR // 16):
        pend.append(pltpu.async_copy(
            zb, acc.at[pl.ds(s * ZR + zi * 16, 16)], semz))
    for d in pend:
        d.wait()


def _sc_scatter_body(xn, src3, dst3, zhbm, out, acc, zb, sl0, sl1, dl0, dl1,
                     r0, r1, sem0, sem1, semi0, semi1, semz):
    c = lax.axis_index("c")
    s = lax.axis_index("s")
    nch = jnp.where(c == 0, NCH0, NCH1)
    g0 = c * F0 + s * nch

    def idx_load(j, sl, dl, semi):
        pltpu.async_copy(src3.at[g0 + j], sl, semi)
        pltpu.async_copy(dst3.at[g0 + j], dl, semi)

    def idx_wait(j, sl, dl, semi):
        pltpu.make_async_copy(src3.at[g0 + j], sl, semi).wait()
        pltpu.make_async_copy(dst3.at[g0 + j], dl, semi).wait()

    idx_load(0, sl0, dl0, semi0)
    idx_load(1, sl1, dl1, semi1)
    _zero_acc(acc, zb, zhbm, semz, s)
    plsc.subcore_barrier()

    idx_wait(0, sl0, dl0, semi0)
    pltpu.async_copy(xn.at[sl0], r0, sem0)

    def step(it, carry):
        i0 = it * 2
        idx_wait(i0 + 1, sl1, dl1, semi1)
        pltpu.async_copy(xn.at[sl1], r1, sem1)
        pltpu.make_async_copy(xn.at[sl0], r0, sem0).wait()
        pltpu.sync_copy(r0, acc.at[dl0], add=True)

        @pl.when(i0 + 2 < nch)
        def _():
            idx_load(i0 + 2, sl0, dl0, semi0)

        pltpu.make_async_copy(xn.at[sl1], r1, sem1).wait()
        pltpu.sync_copy(r1, acc.at[dl1], add=True)

        @pl.when(i0 + 3 < nch)
        def _():
            idx_load(i0 + 3, sl1, dl1, semi1)

        @pl.when(i0 + 2 < nch)
        def _():
            idx_wait(i0 + 2, sl0, dl0, semi0)
            pltpu.async_copy(xn.at[sl0], r0, sem0)

        return carry

    lax.fori_loop(0, nch // 2, step, 0)
    plsc.subcore_barrier()
    pltpu.sync_copy(acc.at[pl.ds(s * OUTR, OUTR)],
                    out.at[c, pl.ds(s * OUTR, OUTR)])


def _sc_scatter(xn, src3, dst3, zhbm):
    return pl.kernel(
        _sc_scatter_body,
        out_type=jax.ShapeDtypeStruct((NC, NP, H), jnp.float32),
        mesh=plsc.VectorSubcoreMesh(**_MESH),
        scratch_types=[
            pltpu.VMEM_SHARED((ACC_ROWS, H), jnp.float32),
            pltpu.VMEM((16, H), jnp.float32),
            pltpu.VMEM((CH,), jnp.int32),
            pltpu.VMEM((CH,), jnp.int32),
            pltpu.VMEM((CH,), jnp.int32),
            pltpu.VMEM((CH,), jnp.int32),
            pltpu.VMEM((CH, H), jnp.float32),
            pltpu.VMEM((CH, H), jnp.float32),
            pltpu.SemaphoreType.DMA,
            pltpu.SemaphoreType.DMA,
            pltpu.SemaphoreType.DMA,
            pltpu.SemaphoreType.DMA,
            pltpu.SemaphoreType.DMA,
        ],
    )(xn, src3, dst3, zhbm)


def _sc_deg_body(dst3, ones_hbm, zhbm, out, acc, zb, di3, ones_v, semz,
                 semd):
    c = lax.axis_index("c")
    s = lax.axis_index("s")
    w = c * NS + s
    pltpu.sync_copy(dst3.at[pl.ds(w * NCH, NCH)], di3)
    pltpu.sync_copy(ones_hbm, ones_v)
    _zero_acc(acc, zb, zhbm, semz, s)
    plsc.subcore_barrier()

    def step(it, carry):
        i0 = it * 2
        pltpu.async_copy(ones_v, acc.at[di3.at[i0]], semd, add=True)
        pltpu.async_copy(ones_v, acc.at[di3.at[i0 + 1]], semd, add=True)
        pltpu.make_async_copy(ones_v, acc.at[di3.at[i0]], semd).wait()
        pltpu.make_async_copy(ones_v, acc.at[di3.at[i0 + 1]], semd).wait()
        return carry

    lax.fori_loop(0, NCH // 2, step, 0)
    plsc.subcore_barrier()
    pltpu.sync_copy(acc.at[pl.ds(s * OUTR, OUTR)],
                    out.at[c, pl.ds(s * OUTR, OUTR)])


def _sc_deg(dst3, ones_hbm, zhbm):
    return pl.kernel(
        _sc_deg_body,
        out_type=jax.ShapeDtypeStruct((NC, NP, H), jnp.float32),
        mesh=plsc.VectorSubcoreMesh(**_MESH),
        scratch_types=[
            pltpu.VMEM_SHARED((ACC_ROWS, H), jnp.float32),
            pltpu.VMEM((16, H), jnp.float32),
            pltpu.VMEM((NCH, CH), jnp.int32),
            pltpu.VMEM((CH, H), jnp.float32),
            pltpu.SemaphoreType.DMA,
            pltpu.SemaphoreType.DMA,
        ],
    )(dst3, ones_hbm, zhbm)


def _mm2_body(x_ref, ws_ref, wn_ref, xs_ref, xn_ref):
    x = x_ref[...]
    xs_ref[...] = jnp.dot(x, ws_ref[...], preferred_element_type=jnp.float32)
    xn_ref[...] = jnp.dot(x, wn_ref[...], preferred_element_type=jnp.float32)


def _mm2(x, ws, wn):
    n, k = x.shape
    return pl.pallas_call(
        _mm2_body,
        grid=(n // BM,),
        in_specs=[
            pl.BlockSpec((BM, k), lambda i: (i, 0)),
            pl.BlockSpec((k, H), lambda i: (0, 0)),
            pl.BlockSpec((k, H), lambda i: (0, 0)),
        ],
        out_specs=[
            pl.BlockSpec((BM, H), lambda i: (i, 0)),
            pl.BlockSpec((BM, H), lambda i: (i, 0)),
        ],
        out_shape=[jax.ShapeDtypeStruct((n, H), jnp.float32)] * 2,
    )(x, ws, wn)


def _ew_math(xs_ref, p_ref, degp_ref, b_ref, a_ref, g_ref, be_ref):
    deg = degp_ref[0, :, 0:1] + degp_ref[1, :, 0:1]
    inv = 1.0 / jnp.maximum(deg, 1.0)
    t = xs_ref[...] + (p_ref[0, :, :] + p_ref[1, :, :]) * inv + b_ref[...]
    t = jnp.where(t >= 0.0, t, a_ref[...] * t)
    return g_ref[...] * t * RSQ + be_ref[...]


def _ew_body(xs_ref, p_ref, degp_ref, b_ref, a_ref, g_ref, be_ref, h_ref):
    h_ref[...] = _ew_math(xs_ref, p_ref, degp_ref, b_ref, a_ref, g_ref,
                          be_ref)


def _ewmm_body(xs_ref, p_ref, degp_ref, b_ref, a_ref, g_ref, be_ref, ws_ref,
               wn_ref, h_ref, xs2_ref, xn2_ref):
    h = _ew_math(xs_ref, p_ref, degp_ref, b_ref, a_ref, g_ref, be_ref)
    h_ref[...] = h
    xs2_ref[...] = jnp.dot(h, ws_ref[...], preferred_element_type=jnp.float32)
    xn2_ref[...] = jnp.dot(h, wn_ref[...], preferred_element_type=jnp.float32)


def _ewmm(xs, p, degp, b, a, g, be, ws, wn):
    return pl.pallas_call(
        _ewmm_body,
        grid=(NP // BM,),
        in_specs=[
            pl.BlockSpec((BM, H), lambda i: (i, 0)),
            pl.BlockSpec((NC, BM, H), lambda i: (0, i, 0)),
            pl.BlockSpec((NC, BM, H), lambda i: (0, i, 0)),
            pl.BlockSpec((1, H), lambda i: (0, 0)),
            pl.BlockSpec((1, H), lambda i: (0, 0)),
            pl.BlockSpec((1, H), lambda i: (0, 0)),
            pl.BlockSpec((1, H), lambda i: (0, 0)),
            pl.BlockSpec((H, H), lambda i: (0, 0)),
            pl.BlockSpec((H, H), lambda i: (0, 0)),
        ],
        out_specs=[
            pl.BlockSpec((BM, H), lambda i: (i, 0)),
            pl.BlockSpec((BM, H), lambda i: (i, 0)),
            pl.BlockSpec((BM, H), lambda i: (i, 0)),
        ],
        out_shape=[jax.ShapeDtypeStruct((NP, H), jnp.float32)] * 3,
    )(xs, p, degp, b, a, g, be, ws, wn)


def _ew(xs, p, degp, b, a, g, be):
    return pl.pallas_call(
        _ew_body,
        grid=(NP // BM,),
        in_specs=[
            pl.BlockSpec((BM, H), lambda i: (i, 0)),
            pl.BlockSpec((NC, BM, H), lambda i: (0, i, 0)),
            pl.BlockSpec((NC, BM, H), lambda i: (0, i, 0)),
            pl.BlockSpec((1, H), lambda i: (0, 0)),
            pl.BlockSpec((1, H), lambda i: (0, 0)),
            pl.BlockSpec((1, H), lambda i: (0, 0)),
            pl.BlockSpec((1, H), lambda i: (0, 0)),
        ],
        out_specs=pl.BlockSpec((BM, H), lambda i: (i, 0)),
        out_shape=jax.ShapeDtypeStruct((NP, H), jnp.float32),
    )(xs, p, degp, b, a, g, be)


def _graph_body(gid_ref, h1_ref, h2_ref, h3_ref, h4_ref, wl_ref, bl_ref,
                out_ref, gs_scr):
    j = pl.program_id(0)

    @pl.when(j == 0)
    def _():
        gs_scr[...] = jnp.zeros_like(gs_scr)

    gid = gid_ref[0, 0, :]
    onehot = (lax.broadcasted_iota(jnp.int32, (128, BG), 0)
              == gid[None, :]).astype(jnp.float32)
    hcat = jnp.concatenate(
        [h1_ref[...], h2_ref[...], h3_ref[...], h4_ref[...],
         jnp.ones((BG, 128), jnp.float32)], axis=1)
    gs_scr[...] += jnp.dot(onehot, hcat, preferred_element_type=jnp.float32)

    @pl.when(j == pl.num_programs(0) - 1)
    def _():
        cnt = gs_scr[:, 512:513]
        gvec = gs_scr[:, :512] / jnp.maximum(cnt, 1.0)
        out_ref[...] = (jnp.dot(gvec, wl_ref[...],
                                preferred_element_type=jnp.float32)
                        + bl_ref[...])


def _graph(gid, h1, h2, h3, h4, wl, bl):
    return pl.pallas_call(
        _graph_body,
        grid=(NP // BG,),
        in_specs=[
            pl.BlockSpec((1, 1, BG), lambda i: (i, 0, 0)),
            pl.BlockSpec((BG, H), lambda i: (i, 0)),
            pl.BlockSpec((BG, H), lambda i: (i, 0)),
            pl.BlockSpec((BG, H), lambda i: (i, 0)),
            pl.BlockSpec((BG, H), lambda i: (i, 0)),
            pl.BlockSpec((4 * H, WIN * H), lambda i: (0, 0)),
            pl.BlockSpec((1, WIN * H), lambda i: (0, 0)),
        ],
        out_specs=pl.BlockSpec((128, WIN * H), lambda i: (0, 0)),
        out_shape=jax.ShapeDtypeStruct((128, WIN * H), jnp.float32),
        scratch_shapes=[pltpu.VMEM((128, 5 * H), jnp.float32)],
    )(gid, h1, h2, h3, h4, wl, bl)


def kernel(x, edge_index, graph_ids, Ws0, Wn0, b0, a0, g0, be0, Ws1, Wn1, b1,
           a1, g1, be1, Ws2, Wn2, b2, a2, g2, be2, Ws3, Wn3, b3, a3, g3, be3,
           Wl, bl):
    f32 = jnp.float32
    src = edge_index[0]
    dst = edge_index[1]
    src3 = jnp.concatenate(
        [src, jnp.full((EP - E,), N - 1, jnp.int32)]).reshape(EP // CH, CH)
    dst3 = jnp.concatenate(
        [dst, jnp.full((EP - E,), NP, jnp.int32)]).reshape(EP // CH, CH)
    xp = jnp.pad(x, ((0, NP - N), (0, 0)))
    gid = jnp.concatenate(
        [graph_ids, jnp.full((NP - N,), 127, jnp.int32)]).reshape(
            NP // BG, 1, BG)
    zhbm = jnp.zeros((16, H), f32)
    ones_hbm = jnp.ones((CH, H), f32)

    degp = _sc_deg(dst3, ones_hbm, zhbm)

    params = [(Ws0, Wn0, b0, a0, g0, be0), (Ws1, Wn1, b1, a1, g1, be1),
              (Ws2, Wn2, b2, a2, g2, be2), (Ws3, Wn3, b3, a3, g3, be3)]
    hs = []
    xs, xn = _mm2(xp, params[0][0], params[0][1])
    for li, (ws, wn, b, a, g, be) in enumerate(params):
        p = _sc_scatter(xn, src3, dst3, zhbm)
        args = (xs, p, degp, b.reshape(1, H), a.reshape(1, H),
                g.reshape(1, H), be.reshape(1, H))
        if li < 3:
            h, xs, xn = _ewmm(*args, params[li + 1][0], params[li + 1][1])
        else:
            h = _ew(*args)
        hs.append(h)

    outp = _graph(gid, hs[0], hs[1], hs[2], hs[3], Wl,
                  bl.reshape(1, WIN * H))
    return outp[:B].reshape(B, WIN, H)
